# baseline (speedup 1.0000x reference)
import functools
import jax
import jax.numpy as jnp
from jax import lax
from jax.experimental import pallas as pl
from jax.experimental.pallas import tpu as pltpu
from jax.experimental.pallas import tpu_sc as plsc

_SCALE = 8.0
_PADW = 128
_CB = 8
_S1 = 128


def _scale_pad_body(t_ref, o_ref):
    t = t_ref[...]
    o_ref[...] = jnp.concatenate(
        [t * _SCALE, jnp.zeros_like(t)], axis=1)


def _scale_pad_table(table):
    v, d = table.shape
    br = 1024
    grid = (v + br - 1) // br
    return pl.pallas_call(
        _scale_pad_body,
        out_shape=jax.ShapeDtypeStruct((v, 2 * d), table.dtype),
        grid=(grid,),
        in_specs=[pl.BlockSpec((br, d), lambda i: (i, 0))],
        out_specs=pl.BlockSpec((br, 2 * d), lambda i: (i, 0)),
    )(table)


@functools.cache
def _make_gather(v, d, bt, s):
    info = plsc.get_sparse_core_info()
    nw = info.num_cores * info.num_subcores
    nc = info.num_cores
    bat_per_w = bt // nw
    n_chunks = bat_per_w // _CB
    assert bat_per_w % _CB == 0
    s2 = s - _S1
    mesh = plsc.VectorSubcoreMesh(core_axis_name="c", subcore_axis_name="s")

    scratch = (
        [pltpu.VMEM((3, _CB, s), jnp.int32)]
        + [pltpu.VMEM((s, _PADW), jnp.float32) for _ in range(2)]
        + [pltpu.VMEM((s, d), jnp.float32) for _ in range(2)]
        + [pltpu.SemaphoreType.DMA for _ in range(2)]
        + [pltpu.SemaphoreType.DMA for _ in range(2)]
        + [pltpu.SemaphoreType.DMA((3,))]
    )

    @functools.partial(
        pl.kernel,
        mesh=mesh,
        out_type=jax.ShapeDtypeStruct((bt, s, d), jnp.float32),
        scratch_types=scratch,
    )
    def gather_kernel(table_hbm, idx_hbm, out_hbm, idx_v, *bufs_and_sems):
        rows = bufs_and_sems[0:2]
        comp = bufs_and_sems[2:4]
        sem_g = bufs_and_sems[4:6]
        sem_w = bufs_and_sems[6:8]
        sem_i = bufs_and_sems[8]
        wid = lax.axis_index("s") * nc + lax.axis_index("c")
        bat0 = wid * bat_per_w

        def idx_chunk_copy(c, ib):
            return pltpu.make_async_copy(
                idx_hbm.at[pl.ds(bat0 + c * _CB, _CB)],
                idx_v.at[ib],
                sem_i.at[ib],
            )

        def gather_copies(ci, r, p):
            return [
                pltpu.make_async_copy(
                    table_hbm.at[idx_v.at[ci, r, pl.ds(0, _S1)]],
                    rows[p].at[pl.ds(0, _S1)],
                    sem_g[p],
                ),
                pltpu.make_async_copy(
                    table_hbm.at[idx_v.at[ci, r, pl.ds(_S1, s2)]],
                    rows[p].at[pl.ds(_S1, s2)],
                    sem_g[p],
                ),
            ]

        def fire_gathers(ci, r, p):
            for cp in gather_copies(ci, r, p):
                cp.start()

        def drain_gathers(ci, r, p):
            for cp in gather_copies(ci, r, p):
                cp.wait()

        def write_copy(g, cc):
            return pltpu.make_async_copy(
                comp[cc],
                out_hbm.at[bat0 + g],
                sem_w[cc],
            )

        def repack(src, dst):
            @plsc.parallel_loop(0, s, unroll=8)
            def _(r):
                for q in range(d // 16):
                    dst[r, pl.ds(q * 16, 16)] = src[r, pl.ds(q * 16, 16)]

        idx_chunk_copy(0, 0).start()
        idx_chunk_copy(0, 0).wait()
        idx_chunk_copy(1, 1).start()
        fire_gathers(0, 0, 0)

        def chunk_fn(c, carry):
            ci = lax.rem(c, 3)
            ci1 = lax.rem(c + 1, 3)

            @pl.when(c + 2 <= n_chunks - 1)
            def _():
                idx_chunk_copy(c + 2, lax.rem(c + 2, 3)).start()

            for bb in range(_CB):
                g = c * _CB + bb
                p = bb % 2
                drain_gathers(ci, bb, p)
                if bb == _CB - 1:
                    @pl.when(c + 1 <= n_chunks - 1)
                    def _(ci1=ci1):
                        idx_chunk_copy(c + 1, ci1).wait()
                        fire_gathers(ci1, 0, 1 - p)
                else:
                    fire_gathers(ci, bb + 1, 1 - p)
                @pl.when(g >= 2)
                def _(p=p, g=g):
                    write_copy(g - 2, p).wait()
                repack(rows[p], comp[p])
                write_copy(g, p).start()
            return carry

        lax.fori_loop(0, n_chunks, chunk_fn, 0)

        last = bat_per_w - 1
        write_copy(last - 1, (last - 1) % 2).wait()
        write_copy(last, last % 2).wait()

    return gather_kernel


def kernel(x, table):
    v, d = table.shape
    bt, s = x.shape
    scaled = _scale_pad_table(table)
    return _make_gather(v, d, bt, s)(scaled, x)
